# Initial kernel scaffold; baseline (speedup 1.0000x reference)
import jax
import jax.numpy as jnp
from jax.experimental import pallas as pl


def kernel(positions, table):
    raise NotImplementedError("write your pallas kernel here")


# speedup vs baseline: 4.5976x; 4.5976x over previous
import functools

import jax
import jax.numpy as jnp
from jax import lax
from jax.experimental import pallas as pl
from jax.experimental.pallas import tpu as pltpu
from jax.experimental.pallas import tpu_sc as plsc


def kernel(positions, table):
    Bb, Ll = positions.shape
    V, D = table.shape
    B = Bb * Ll
    info = plsc.get_sparse_core_info()
    NC, NS = info.num_cores, info.num_subcores
    nw = NC * NS
    C = 128
    b_per_w = B // nw
    n_chunks = b_per_w // C
    assert b_per_w * nw == B and n_chunks * C == b_per_w

    idx = positions.reshape(B).astype(jnp.int32)
    mesh = plsc.VectorSubcoreMesh(core_axis_name="c", subcore_axis_name="s")

    @functools.partial(
        pl.kernel,
        mesh=mesh,
        out_type=jax.ShapeDtypeStruct((B, D), jnp.float32),
        scratch_types=[
            pltpu.VMEM((C,), jnp.int32),
            pltpu.VMEM((C, D), jnp.float32),
            pltpu.SemaphoreType.DMA,
        ],
    )
    def gather_kernel(idx_hbm, table_hbm, out_hbm, idx_v, rows_v, sem):
        wid = lax.axis_index("s") * NC + lax.axis_index("c")
        base = wid * b_per_w

        def body(i, carry):
            off = base + i * C
            pltpu.sync_copy(idx_hbm.at[pl.ds(off, C)], idx_v)
            pltpu.async_copy(table_hbm.at[idx_v], rows_v, sem).wait()
            pltpu.sync_copy(rows_v, out_hbm.at[pl.ds(off, C)])
            return carry

        lax.fori_loop(0, n_chunks, body, 0)

    out = gather_kernel(idx, table)
    return out.reshape(Bb, Ll, D)


# speedup vs baseline: 6.6781x; 1.4525x over previous
import functools

import jax
import jax.numpy as jnp
from jax import lax
from jax.experimental import pallas as pl
from jax.experimental.pallas import tpu as pltpu
from jax.experimental.pallas import tpu_sc as plsc


def kernel(positions, table):
    Bb, Ll = positions.shape
    V, D = table.shape
    B = Bb * Ll
    info = plsc.get_sparse_core_info()
    NC, NS = info.num_cores, info.num_subcores
    nw = NC * NS
    C = 128
    NBUF = 5
    b_per_w = B // nw
    n_chunks = b_per_w // C
    assert b_per_w * nw == B and n_chunks * C == b_per_w
    assert n_chunks % NBUF == 0

    idx = positions.reshape(nw, n_chunks, C).astype(jnp.int32)
    mesh = plsc.VectorSubcoreMesh(core_axis_name="c", subcore_axis_name="s")

    @functools.partial(
        pl.kernel,
        mesh=mesh,
        out_type=jax.ShapeDtypeStruct((B, D), jnp.float32),
        scratch_types=[
            pltpu.VMEM((n_chunks, C), jnp.int32),
            pltpu.VMEM((NBUF, C, D), jnp.float32),
            pltpu.SemaphoreType.DMA((NBUF,)),
            pltpu.SemaphoreType.DMA((NBUF,)),
        ],
    )
    def gather_kernel(idx_hbm, table_hbm, out_hbm, idx_v, rows, gsem, ssem):
        wid = lax.axis_index("s") * NC + lax.axis_index("c")
        base = wid * b_per_w
        pltpu.sync_copy(idx_hbm.at[wid], idx_v)

        def gather_start(chunk, buf):
            pltpu.async_copy(
                table_hbm.at[idx_v.at[chunk]], rows.at[buf], gsem.at[buf]
            )

        def gather_wait(chunk, buf):
            pltpu.make_async_copy(
                table_hbm.at[idx_v.at[chunk]], rows.at[buf], gsem.at[buf]
            ).wait()

        def store_start(chunk, buf):
            pltpu.async_copy(
                rows.at[buf], out_hbm.at[pl.ds(base + chunk * C, C)], ssem.at[buf]
            )

        def store_wait(chunk, buf):
            pltpu.make_async_copy(
                rows.at[buf], out_hbm.at[pl.ds(base + chunk * C, C)], ssem.at[buf]
            ).wait()

        for b in range(NBUF - 1):
            gather_start(b, b)

        def body(g, carry):
            i0 = g * NBUF
            for b in range(NBUF):
                i = i0 + b
                bprev = (b - 1) % NBUF
                gather_wait(i, b)
                store_start(i, b)
                j = i + NBUF - 1
                if b == 0:
                    pl.when(g > 0)(lambda: store_wait(i - 1, bprev))
                else:
                    store_wait(i - 1, bprev)
                pl.when(j < n_chunks)(lambda: gather_start(j, bprev))
            return carry

        lax.fori_loop(0, n_chunks // NBUF, body, 0)
        store_wait(n_chunks - 1, (n_chunks - 1) % NBUF)

    out = gather_kernel(idx, table)
    return out.reshape(Bb, Ll, D)


# speedup vs baseline: 11.8757x; 1.7783x over previous
import functools

import jax
import jax.numpy as jnp
from jax import lax
from jax.experimental import pallas as pl
from jax.experimental.pallas import tpu as pltpu
from jax.experimental.pallas import tpu_sc as plsc


def kernel(positions, table):
    Bb, Ll = positions.shape
    V, D = table.shape
    B = Bb * Ll
    info = plsc.get_sparse_core_info()
    NC, NS = info.num_cores, info.num_subcores
    nw = NC * NS
    C = 128
    NBUF = 5
    b_per_w = B // nw
    n_chunks = b_per_w // C
    assert b_per_w * nw == B and n_chunks * C == b_per_w
    assert n_chunks % NBUF == 0

    idx = positions.reshape(nw, n_chunks, C).astype(jnp.int32)
    mesh = plsc.VectorSubcoreMesh(core_axis_name="c", subcore_axis_name="s")

    @functools.partial(
        pl.kernel,
        mesh=mesh,
        out_type=jax.ShapeDtypeStruct((B, D), jnp.float32),
        scratch_types=[
            pltpu.VMEM((n_chunks, C), jnp.int32),
            pltpu.VMEM((NBUF, C, D), jnp.float32),
            pltpu.VMEM_SHARED((V, D), jnp.float32),
            pltpu.SemaphoreType.DMA((NBUF,)),
            pltpu.SemaphoreType.DMA((NBUF,)),
        ],
    )
    def gather_kernel(idx_hbm, table_hbm, out_hbm, idx_v, rows, tshared, gsem, ssem):
        sid = lax.axis_index("s")
        wid = sid * NC + lax.axis_index("c")
        base = wid * b_per_w
        pl.when(sid == 0)(lambda: pltpu.sync_copy(table_hbm, tshared))
        pltpu.sync_copy(idx_hbm.at[wid], idx_v)
        plsc.subcore_barrier()

        def gather_start(chunk, buf):
            pltpu.async_copy(
                tshared.at[idx_v.at[chunk]], rows.at[buf], gsem.at[buf]
            )

        def gather_wait(chunk, buf):
            pltpu.make_async_copy(
                tshared.at[idx_v.at[chunk]], rows.at[buf], gsem.at[buf]
            ).wait()

        def store_start(chunk, buf):
            pltpu.async_copy(
                rows.at[buf], out_hbm.at[pl.ds(base + chunk * C, C)], ssem.at[buf]
            )

        def store_wait(chunk, buf):
            pltpu.make_async_copy(
                rows.at[buf], out_hbm.at[pl.ds(base + chunk * C, C)], ssem.at[buf]
            ).wait()

        for b in range(NBUF - 1):
            gather_start(b, b)

        def body(g, carry):
            i0 = g * NBUF
            for b in range(NBUF):
                i = i0 + b
                bprev = (b - 1) % NBUF
                gather_wait(i, b)
                store_start(i, b)
                j = i + NBUF - 1
                if b == 0:
                    pl.when(g > 0)(lambda: store_wait(i - 1, bprev))
                else:
                    store_wait(i - 1, bprev)
                pl.when(j < n_chunks)(lambda: gather_start(j, bprev))
            return carry

        lax.fori_loop(0, n_chunks // NBUF, body, 0)
        store_wait(n_chunks - 1, (n_chunks - 1) % NBUF)

    out = gather_kernel(idx, table)
    return out.reshape(Bb, Ll, D)
